# baseline (speedup 1.0000x reference)
import functools

import numpy as np
import jax
import jax.numpy as jnp
from jax import lax
from jax.experimental import pallas as pl
from jax.experimental.pallas import tpu as pltpu
from jax.experimental.pallas import tpu_sc as plsc

_SIZE_VERTEX = 100000
_D = 64


def _build_leaves(size_vertex):
    leaf = []

    def rec(tl, tr, v):
        if tl == tr:
            leaf.append(v)
            return
        tm = (tl + tr) >> 1
        rec(tl, tm, 2 * v)
        rec(tm + 1, tr, 2 * v + 1)

    rec(1, size_vertex + 1, 1)
    return leaf


_LEAF = np.asarray(_build_leaves(_SIZE_VERTEX), dtype=np.int32)
_MAX_BITS = int(_LEAF.max()).bit_length()
_NLEV = _MAX_BITS - 1
_NPAD = 32

_PERM_TBL = np.concatenate(
    [np.arange(16, dtype=np.int32) ^ (1 << k) for k in range(4)])
_OH_TBL = np.eye(16, dtype=np.float32).ravel()

_mesh = plsc.VectorSubcoreMesh(core_axis_name="c", subcore_axis_name="s")


@functools.partial(
    pl.kernel,
    out_type=jax.ShapeDtypeStruct((16,), jnp.float32),
    mesh=_mesh,
    compiler_params=pltpu.CompilerParams(use_tc_tiling_on_sc=False),
    scratch_types=[
        pltpu.VMEM((_NPAD,), jnp.int32),
        pltpu.VMEM((1,), jnp.int32),
        pltpu.VMEM((1, _D), jnp.float32),
        pltpu.VMEM((_NPAD, _D), jnp.float32),
        pltpu.VMEM((_NPAD,), jnp.float32),
        pltpu.VMEM((_NPAD,), jnp.float32),
        pltpu.VMEM((64,), jnp.int32),
        pltpu.VMEM((256,), jnp.float32),
        pltpu.VMEM((16,), jnp.float32),
        pltpu.SemaphoreType.DMA,
    ],
)
def _hs_path_kernel(w_hbm, enc_hbm, parents_hbm, vj_hbm, mult_hbm, valid_hbm,
                    perm_hbm, oh_hbm, out_hbm, pidx_v, vj_v, h_v, rows_v,
                    mult_v, valid_v, perm_v, oh_v, out_v, sem):
    cid = lax.axis_index("c")
    sid = lax.axis_index("s")

    @pl.when(jnp.logical_and(cid == 0, sid == 0))
    def _():
        pltpu.sync_copy(parents_hbm, pidx_v)
        pltpu.sync_copy(vj_hbm, vj_v)
        pltpu.sync_copy(mult_hbm, mult_v)
        pltpu.sync_copy(valid_hbm, valid_v)
        pltpu.sync_copy(perm_hbm, perm_v)
        pltpu.sync_copy(oh_hbm, oh_v)
        cp_h = pltpu.async_copy(enc_hbm.at[vj_v], h_v, sem)
        cp_w = pltpu.async_copy(w_hbm.at[pidx_v], rows_v, sem)
        cp_h.wait()
        cp_w.wait()

        perms = [perm_v[pl.ds(16 * k, 16)] for k in range(4)]
        ohs = [oh_v[pl.ds(16 * r, 16)] for r in range(16)]
        h_chunks = [h_v[0, pl.ds(16 * k, 16)] for k in range(_D // 16)]

        fprod = None
        for half in range(_NPAD // 16):
            dots = None
            for r16 in range(16):
                r = 16 * half + r16
                acc = rows_v[r, pl.ds(0, 16)] * h_chunks[0]
                for k in range(1, _D // 16):
                    acc = acc + rows_v[r, pl.ds(16 * k, 16)] * h_chunks[k]
                for k in range(4):
                    acc = acc + acc.at[perms[k]].get(mode="promise_in_bounds")
                term = ohs[r16] * acc
                dots = term if dots is None else dots + term
            m = mult_v[pl.ds(16 * half, 16)]
            va = valid_v[pl.ds(16 * half, 16)]
            f = 1.0 / (1.0 + jnp.exp(-(m * dots)))
            f = va * f + (1.0 - va)
            fprod = f if fprod is None else fprod * f

        p = fprod[0]
        for i in range(1, 16):
            p = p * fprod[i]
        out_v[pl.ds(0, 16)] = jnp.broadcast_to(p, (16,))
        pltpu.sync_copy(out_v, out_hbm)


def kernel(encoder, W, v_i, v_j):
    new_node = jnp.take(jnp.asarray(_LEAF), jnp.asarray(v_i, jnp.int32))
    lev = jnp.arange(_NLEV, dtype=jnp.int32)
    s = (_MAX_BITS - 2) - lev
    parent = lax.shift_right_logical(new_node, s + 1)
    child = lax.shift_right_logical(new_node, s)
    mult = jnp.where(child % 2 == 0, 1.0, -1.0).astype(jnp.float32)
    valid = (parent >= 2).astype(jnp.float32)
    pad = _NPAD - _NLEV
    parents = jnp.concatenate([parent, jnp.zeros((pad,), jnp.int32)])
    mult = jnp.concatenate([mult, jnp.zeros((pad,), jnp.float32)])
    valid = jnp.concatenate([valid, jnp.zeros((pad,), jnp.float32)])
    vj = jnp.reshape(jnp.asarray(v_j, jnp.int32), (1,))
    out = _hs_path_kernel(W, encoder, parents, vj, mult, valid,
                          jnp.asarray(_PERM_TBL), jnp.asarray(_OH_TBL))
    return out[0:1]


# speedup vs baseline: 6.3150x; 6.3150x over previous
import functools

import numpy as np
import jax
import jax.numpy as jnp
from jax import lax
from jax.experimental import pallas as pl
from jax.experimental.pallas import tpu as pltpu
from jax.experimental.pallas import tpu_sc as plsc

_SIZE_VERTEX = 100000
_D = 64


def _build_leaves(size_vertex):
    leaf = []

    def rec(tl, tr, v):
        if tl == tr:
            leaf.append(v)
            return
        tm = (tl + tr) >> 1
        rec(tl, tm, 2 * v)
        rec(tm + 1, tr, 2 * v + 1)

    rec(1, size_vertex + 1, 1)
    return leaf


_LEAF = np.asarray(_build_leaves(_SIZE_VERTEX), dtype=np.int32)
_MAX_BITS = int(_LEAF.max()).bit_length()
_NLEV = _MAX_BITS - 1
_NPAD = 32
_NSUB = 16

_PERM_TBL = np.zeros((4, 128), dtype=np.int32)
for _k in range(4):
    _PERM_TBL[_k, :16] = np.arange(16, dtype=np.int32) ^ (1 << _k)

_mesh = plsc.VectorSubcoreMesh(core_axis_name="c", subcore_axis_name="s")


@functools.partial(
    pl.kernel,
    out_type=jax.ShapeDtypeStruct((16,), jnp.float32),
    mesh=_mesh,
    compiler_params=pltpu.CompilerParams(use_tc_tiling_on_sc=True),
    scratch_types=[
        pltpu.VMEM((8, 128), jnp.int32),
        pltpu.VMEM((8, 128), jnp.float32),
        pltpu.VMEM((_D, 128), jnp.float32),
        pltpu.VMEM((_D, 128), jnp.float32),
        pltpu.VMEM((_D, 128), jnp.float32),
        pltpu.VMEM((16,), jnp.float32),
        pltpu.VMEM((_NPAD, 16), jnp.float32),
        pltpu.VMEM_SHARED((_NPAD, 16), jnp.float32),
        pltpu.SemaphoreType.DMA,
    ],
)
def _hs_path_kernel(wt_hbm, enct_hbm, idx3_hbm, par3_hbm, out_hbm,
                    idxt_v, part_v, w0_v, w1_v, e_v, fac_v, allfac_v,
                    shared, sem):
    cid = lax.axis_index("c")
    sid = lax.axis_index("s")

    @pl.when(cid == 0)
    def _():
        pltpu.sync_copy(idx3_hbm.at[sid], idxt_v)
        pltpu.sync_copy(par3_hbm.at[sid], part_v)
        ivec = idxt_v[0, pl.ds(0, 16)]
        wb0 = pl.multiple_of(ivec[0], 128)
        wb1 = pl.multiple_of(ivec[1], 128)
        eb = pl.multiple_of(ivec[2], 128)
        perms = [idxt_v[1 + k, pl.ds(0, 16)] for k in range(4)]

        cp0 = pltpu.async_copy(wt_hbm.at[:, pl.ds(wb0, 128)], w0_v, sem)
        cp1 = pltpu.async_copy(wt_hbm.at[:, pl.ds(wb1, 128)], w1_v, sem)
        cpe = pltpu.async_copy(enct_hbm.at[:, pl.ds(eb, 128)], e_v, sem)
        cp0.wait()
        cp1.wait()
        cpe.wait()

        oh0 = [part_v[0, pl.ds(16 * j, 16)] for j in range(8)]
        oh1 = [part_v[1, pl.ds(16 * j, 16)] for j in range(8)]
        ohh = [part_v[2, pl.ds(16 * j, 16)] for j in range(8)]

        d0 = None
        d1 = None
        for c in range(_D):
            hm = e_v[c, pl.ds(0, 16)] * ohh[0]
            for j in range(1, 8):
                hm = hm + e_v[c, pl.ds(16 * j, 16)] * ohh[j]
            for k in range(4):
                hm = hm + hm.at[perms[k]].get(mode="promise_in_bounds")
            wc0 = w0_v[c, pl.ds(0, 16)] * oh0[0]
            wc1 = w1_v[c, pl.ds(0, 16)] * oh1[0]
            for j in range(1, 8):
                wc0 = wc0 + w0_v[c, pl.ds(16 * j, 16)] * oh0[j]
                wc1 = wc1 + w1_v[c, pl.ds(16 * j, 16)] * oh1[j]
            t0 = hm * wc0
            t1 = hm * wc1
            d0 = t0 if d0 is None else d0 + t0
            d1 = t1 if d1 is None else d1 + t1
        for k in range(4):
            d0 = d0 + d0.at[perms[k]].get(mode="promise_in_bounds")
            d1 = d1 + d1.at[perms[k]].get(mode="promise_in_bounds")

        pvec = part_v[3, pl.ds(0, 16)]
        m0, va0, m1, va1 = pvec[0], pvec[1], pvec[2], pvec[3]
        f0 = va0 * (1.0 / (1.0 + jnp.exp(-(m0 * d0)))) + (1.0 - va0)
        f1 = va1 * (1.0 / (1.0 + jnp.exp(-(m1 * d1)))) + (1.0 - va1)

        fac_v[pl.ds(0, 16)] = f0
        pltpu.sync_copy(fac_v, shared.at[sid])
        fac_v[pl.ds(0, 16)] = f1
        pltpu.sync_copy(fac_v, shared.at[sid + _NSUB])
        plsc.subcore_barrier()

        @pl.when(sid == 0)
        def _():
            pltpu.sync_copy(shared, allfac_v)
            p = allfac_v[0, pl.ds(0, 16)][0]
            for r in range(1, _NPAD):
                p = p * allfac_v[r, pl.ds(0, 16)][0]
            fac_v[pl.ds(0, 16)] = jnp.broadcast_to(p, (16,))
            pltpu.sync_copy(fac_v, out_hbm)


def kernel(encoder, W, v_i, v_j):
    new_node = jnp.take(jnp.asarray(_LEAF), jnp.asarray(v_i, jnp.int32))
    lev = jnp.arange(_NLEV, dtype=jnp.int32)
    s = (_MAX_BITS - 2) - lev
    parent = lax.shift_right_logical(new_node, s + 1)
    child = lax.shift_right_logical(new_node, s)
    mult = jnp.where(child % 2 == 0, 1.0, -1.0).astype(jnp.float32)
    valid = (parent >= 2).astype(jnp.float32)
    pad = _NPAD - _NLEV
    parents = jnp.concatenate([parent, jnp.zeros((pad,), jnp.int32)])
    mult = jnp.concatenate([mult, jnp.zeros((pad,), jnp.float32)])
    valid = jnp.concatenate([valid, jnp.zeros((pad,), jnp.float32)])

    vj = jnp.asarray(v_j, jnp.int32)
    wbase = (parents >> 7) << 7
    wlane = parents & 127
    ebase = (vj >> 7) << 7
    elane = vj & 127

    oh = jax.nn.one_hot(wlane, 128, dtype=jnp.float32)
    ohh = jax.nn.one_hot(elane, 128, dtype=jnp.float32)

    idx_row0 = jnp.zeros((_NSUB, 128), jnp.int32)
    idx_row0 = idx_row0.at[:, 0].set(wbase[:_NSUB])
    idx_row0 = idx_row0.at[:, 1].set(wbase[_NSUB:])
    idx_row0 = idx_row0.at[:, 2].set(ebase)
    perm_rows = jnp.broadcast_to(
        jnp.asarray(_PERM_TBL)[None], (_NSUB, 4, 128))
    idx3 = jnp.concatenate(
        [idx_row0[:, None, :], perm_rows,
         jnp.zeros((_NSUB, 3, 128), jnp.int32)], axis=1)

    par_row3 = jnp.zeros((_NSUB, 128), jnp.float32)
    par_row3 = par_row3.at[:, 0].set(mult[:_NSUB])
    par_row3 = par_row3.at[:, 1].set(valid[:_NSUB])
    par_row3 = par_row3.at[:, 2].set(mult[_NSUB:])
    par_row3 = par_row3.at[:, 3].set(valid[_NSUB:])
    par3 = jnp.concatenate(
        [oh[:_NSUB][:, None, :], oh[_NSUB:][:, None, :],
         jnp.broadcast_to(ohh[None, None, :], (_NSUB, 1, 128)),
         par_row3[:, None, :],
         jnp.zeros((_NSUB, 4, 128), jnp.float32)], axis=1)

    out = _hs_path_kernel(W.T, encoder.T, idx3, par3)
    return out[0:1]


# speedup vs baseline: 6.6984x; 1.0607x over previous
import functools

import numpy as np
import jax
import jax.numpy as jnp
from jax import lax
from jax.experimental import pallas as pl
from jax.experimental.pallas import tpu as pltpu
from jax.experimental.pallas import tpu_sc as plsc

_SIZE_VERTEX = 100000
_D = 64


def _build_leaves(size_vertex):
    leaf = []

    def rec(tl, tr, v):
        if tl == tr:
            leaf.append(v)
            return
        tm = (tl + tr) >> 1
        rec(tl, tm, 2 * v)
        rec(tm + 1, tr, 2 * v + 1)

    rec(1, size_vertex + 1, 1)
    return leaf


_LEAF = np.asarray(_build_leaves(_SIZE_VERTEX), dtype=np.int32)
_MAX_BITS = int(_LEAF.max()).bit_length()
_NLEV = _MAX_BITS - 1
_NPAD = 32
_NSUB = 16

_PERM_TBL = np.zeros((4, 128), dtype=np.int32)
for _k in range(4):
    _PERM_TBL[_k, :16] = np.arange(16, dtype=np.int32) ^ (1 << _k)

_mesh = plsc.VectorSubcoreMesh(core_axis_name="c", subcore_axis_name="s")


@functools.partial(
    pl.kernel,
    out_type=jax.ShapeDtypeStruct((16,), jnp.float32),
    mesh=_mesh,
    compiler_params=pltpu.CompilerParams(use_tc_tiling_on_sc=True),
    scratch_types=[
        pltpu.VMEM((8, 128), jnp.int32),
        pltpu.VMEM((8, 128), jnp.float32),
        pltpu.VMEM((_D, 128), jnp.float32),
        pltpu.VMEM((_D, 128), jnp.float32),
        pltpu.VMEM((_D, 128), jnp.float32),
        pltpu.VMEM((16,), jnp.float32),
        pltpu.VMEM((_NPAD, 16), jnp.float32),
        pltpu.VMEM_SHARED((_NPAD, 16), jnp.float32),
        pltpu.SemaphoreType.DMA,
    ],
)
def _hs_path_kernel(wt_hbm, enct_hbm, idx3_hbm, par3_hbm, out_hbm,
                    idxt_v, part_v, w0_v, w1_v, e_v, fac_v, allfac_v,
                    shared, sem):
    cid = lax.axis_index("c")
    sid = lax.axis_index("s")

    @pl.when(cid == 0)
    def _():
        pltpu.sync_copy(idx3_hbm.at[sid], idxt_v)
        pltpu.sync_copy(par3_hbm.at[sid], part_v)
        ivec = idxt_v[0, pl.ds(0, 16)]
        wb0 = pl.multiple_of(ivec[0], 128)
        wb1 = pl.multiple_of(ivec[1], 128)
        eb = pl.multiple_of(ivec[2], 128)
        choh = pl.multiple_of(ivec[3], 16)
        ch0 = pl.multiple_of(ivec[4], 16)
        ch1 = pl.multiple_of(ivec[5], 16)
        lane0 = jnp.broadcast_to(ivec[6], (16,))
        lane1 = jnp.broadcast_to(ivec[7], (16,))
        perms = [idxt_v[1 + k, pl.ds(0, 16)] for k in range(4)]

        cp0 = pltpu.async_copy(wt_hbm.at[:, pl.ds(wb0, 128)], w0_v, sem)
        cp1 = pltpu.async_copy(wt_hbm.at[:, pl.ds(wb1, 128)], w1_v, sem)
        cpe = pltpu.async_copy(enct_hbm.at[:, pl.ds(eb, 128)], e_v, sem)
        cp0.wait()
        cp1.wait()
        cpe.wait()

        d0 = None
        d1 = None
        for c in range(_D):
            hc = e_v[c, pl.ds(choh, 16)]
            w0c = w0_v[c, pl.ds(ch0, 16)]
            w1c = w1_v[c, pl.ds(ch1, 16)]
            w0b = w0c.at[lane0].get(mode="promise_in_bounds")
            w1b = w1c.at[lane1].get(mode="promise_in_bounds")
            t0 = hc * w0b
            t1 = hc * w1b
            d0 = t0 if d0 is None else d0 + t0
            d1 = t1 if d1 is None else d1 + t1

        ohh = part_v[0, pl.ds(0, 16)]
        d0 = d0 * ohh
        d1 = d1 * ohh
        for k in range(4):
            d0 = d0 + d0.at[perms[k]].get(mode="promise_in_bounds")
            d1 = d1 + d1.at[perms[k]].get(mode="promise_in_bounds")

        pvec = part_v[1, pl.ds(0, 16)]
        m0, va0, m1, va1 = pvec[0], pvec[1], pvec[2], pvec[3]
        f0 = va0 * (1.0 / (1.0 + jnp.exp(-(m0 * d0)))) + (1.0 - va0)
        f1 = va1 * (1.0 / (1.0 + jnp.exp(-(m1 * d1)))) + (1.0 - va1)

        fac_v[pl.ds(0, 16)] = f0
        pltpu.sync_copy(fac_v, shared.at[sid])
        fac_v[pl.ds(0, 16)] = f1
        pltpu.sync_copy(fac_v, shared.at[sid + _NSUB])
        plsc.subcore_barrier()

        @pl.when(sid == 0)
        def _():
            pltpu.sync_copy(shared, allfac_v)
            p = allfac_v[0, pl.ds(0, 16)][0]
            for r in range(1, _NPAD):
                p = p * allfac_v[r, pl.ds(0, 16)][0]
            fac_v[pl.ds(0, 16)] = jnp.broadcast_to(p, (16,))
            pltpu.sync_copy(fac_v, out_hbm)


def kernel(encoder, W, v_i, v_j):
    new_node = jnp.take(jnp.asarray(_LEAF), jnp.asarray(v_i, jnp.int32))
    lev = jnp.arange(_NLEV, dtype=jnp.int32)
    s = (_MAX_BITS - 2) - lev
    parent = lax.shift_right_logical(new_node, s + 1)
    child = lax.shift_right_logical(new_node, s)
    mult = jnp.where(child % 2 == 0, 1.0, -1.0).astype(jnp.float32)
    valid = (parent >= 2).astype(jnp.float32)
    pad = _NPAD - _NLEV
    parents = jnp.concatenate([parent, jnp.zeros((pad,), jnp.int32)])
    mult = jnp.concatenate([mult, jnp.zeros((pad,), jnp.float32)])
    valid = jnp.concatenate([valid, jnp.zeros((pad,), jnp.float32)])

    vj = jnp.asarray(v_j, jnp.int32)
    wbase = (parents >> 7) << 7
    wlane = parents & 127
    ebase = (vj >> 7) << 7
    elane = vj & 127
    lhm = elane & 15

    idx_row0 = jnp.zeros((_NSUB, 128), jnp.int32)
    idx_row0 = idx_row0.at[:, 0].set(wbase[:_NSUB])
    idx_row0 = idx_row0.at[:, 1].set(wbase[_NSUB:])
    idx_row0 = idx_row0.at[:, 2].set(ebase)
    idx_row0 = idx_row0.at[:, 3].set((elane >> 4) << 4)
    idx_row0 = idx_row0.at[:, 4].set((wlane[:_NSUB] >> 4) << 4)
    idx_row0 = idx_row0.at[:, 5].set((wlane[_NSUB:] >> 4) << 4)
    idx_row0 = idx_row0.at[:, 6].set(wlane[:_NSUB] & 15)
    idx_row0 = idx_row0.at[:, 7].set(wlane[_NSUB:] & 15)
    perm_rows = jnp.broadcast_to(
        jnp.asarray(_PERM_TBL)[None], (_NSUB, 4, 128))

    ohh16 = jax.nn.one_hot(lhm, 16, dtype=jnp.float32)
    row5 = jnp.zeros((_NSUB, 128), jnp.float32)
    row5 = row5.at[:, 0:16].set(jnp.broadcast_to(ohh16[None], (_NSUB, 16)))
    row6 = jnp.zeros((_NSUB, 128), jnp.float32)
    row6 = row6.at[:, 0].set(mult[:_NSUB])
    row6 = row6.at[:, 1].set(valid[:_NSUB])
    row6 = row6.at[:, 2].set(mult[_NSUB:])
    row6 = row6.at[:, 3].set(valid[_NSUB:])
    par3 = jnp.concatenate(
        [row5[:, None, :], row6[:, None, :],
         jnp.zeros((_NSUB, 6, 128), jnp.float32)], axis=1)
    idx3 = jnp.concatenate(
        [idx_row0[:, None, :], perm_rows,
         jnp.zeros((_NSUB, 3, 128), jnp.int32)], axis=1)

    out = _hs_path_kernel(W.T, encoder.T, idx3, par3)
    return out[0:1]


# speedup vs baseline: 6.9000x; 1.0301x over previous
import functools

import numpy as np
import jax
import jax.numpy as jnp
from jax import lax
from jax.experimental import pallas as pl
from jax.experimental.pallas import tpu as pltpu
from jax.experimental.pallas import tpu_sc as plsc

_SIZE_VERTEX = 100000
_D = 64


def _build_leaves(size_vertex):
    leaf = []

    def rec(tl, tr, v):
        if tl == tr:
            leaf.append(v)
            return
        tm = (tl + tr) >> 1
        rec(tl, tm, 2 * v)
        rec(tm + 1, tr, 2 * v + 1)

    rec(1, size_vertex + 1, 1)
    return leaf


_LEAF = np.asarray(_build_leaves(_SIZE_VERTEX), dtype=np.int32)
_MAX_BITS = int(_LEAF.max()).bit_length()
_NLEV = _MAX_BITS - 1
_NPAD = 32
_NSUB = 16

_PERM_TBL = np.zeros((4, 128), dtype=np.int32)
for _k in range(4):
    _PERM_TBL[_k, :16] = np.arange(16, dtype=np.int32) ^ (1 << _k)

_mesh = plsc.VectorSubcoreMesh(
    core_axis_name="c", subcore_axis_name="s", num_cores=1)


@functools.partial(
    pl.kernel,
    out_type=jax.ShapeDtypeStruct((16,), jnp.float32),
    mesh=_mesh,
    compiler_params=pltpu.CompilerParams(use_tc_tiling_on_sc=True),
    scratch_types=[
        pltpu.VMEM((8, 128), jnp.int32),
        pltpu.VMEM((8, 128), jnp.float32),
        pltpu.VMEM((_D, 128), jnp.float32),
        pltpu.VMEM((_D, 128), jnp.float32),
        pltpu.VMEM((_D, 128), jnp.float32),
        pltpu.VMEM((16,), jnp.float32),
        pltpu.VMEM((_NPAD, 16), jnp.float32),
        pltpu.VMEM_SHARED((_NPAD, 16), jnp.float32),
        pltpu.SemaphoreType.DMA,
    ],
)
def _hs_path_kernel(wt_hbm, enct_hbm, idx3_hbm, par3_hbm, out_hbm,
                    idxt_v, part_v, w0_v, w1_v, e_v, fac_v, allfac_v,
                    shared, sem):
    cid = lax.axis_index("c")
    sid = lax.axis_index("s")

    @pl.when(cid == 0)
    def _():
        pltpu.sync_copy(idx3_hbm.at[sid], idxt_v)
        pltpu.sync_copy(par3_hbm.at[sid], part_v)
        ivec = idxt_v[0, pl.ds(0, 16)]
        wb0 = pl.multiple_of(ivec[0], 128)
        wb1 = pl.multiple_of(ivec[1], 128)
        eb = pl.multiple_of(ivec[2], 128)
        choh = pl.multiple_of(ivec[3], 16)
        ch0 = pl.multiple_of(ivec[4], 16)
        ch1 = pl.multiple_of(ivec[5], 16)
        lane0 = jnp.broadcast_to(ivec[6], (16,))
        lane1 = jnp.broadcast_to(ivec[7], (16,))
        perms = [idxt_v[1 + k, pl.ds(0, 16)] for k in range(4)]

        cp0 = pltpu.async_copy(wt_hbm.at[:, pl.ds(wb0, 128)], w0_v, sem)
        cp1 = pltpu.async_copy(wt_hbm.at[:, pl.ds(wb1, 128)], w1_v, sem)
        cpe = pltpu.async_copy(enct_hbm.at[:, pl.ds(eb, 128)], e_v, sem)
        cp0.wait()
        cp1.wait()
        cpe.wait()

        d0 = None
        d1 = None
        for c in range(_D):
            hc = e_v[c, pl.ds(choh, 16)]
            w0c = w0_v[c, pl.ds(ch0, 16)]
            w1c = w1_v[c, pl.ds(ch1, 16)]
            w0b = w0c.at[lane0].get(mode="promise_in_bounds")
            w1b = w1c.at[lane1].get(mode="promise_in_bounds")
            t0 = hc * w0b
            t1 = hc * w1b
            d0 = t0 if d0 is None else d0 + t0
            d1 = t1 if d1 is None else d1 + t1

        ohh = part_v[0, pl.ds(0, 16)]
        d0 = d0 * ohh
        d1 = d1 * ohh
        for k in range(4):
            d0 = d0 + d0.at[perms[k]].get(mode="promise_in_bounds")
            d1 = d1 + d1.at[perms[k]].get(mode="promise_in_bounds")

        pvec = part_v[1, pl.ds(0, 16)]
        m0, va0, m1, va1 = pvec[0], pvec[1], pvec[2], pvec[3]
        f0 = va0 * (1.0 / (1.0 + jnp.exp(-(m0 * d0)))) + (1.0 - va0)
        f1 = va1 * (1.0 / (1.0 + jnp.exp(-(m1 * d1)))) + (1.0 - va1)

        fac_v[pl.ds(0, 16)] = f0
        pltpu.sync_copy(fac_v, shared.at[sid])
        fac_v[pl.ds(0, 16)] = f1
        pltpu.sync_copy(fac_v, shared.at[sid + _NSUB])
        plsc.subcore_barrier()

        @pl.when(sid == 0)
        def _():
            pltpu.sync_copy(shared, allfac_v)
            p = allfac_v[0, pl.ds(0, 16)][0]
            for r in range(1, _NPAD):
                p = p * allfac_v[r, pl.ds(0, 16)][0]
            fac_v[pl.ds(0, 16)] = jnp.broadcast_to(p, (16,))
            pltpu.sync_copy(fac_v, out_hbm)


def kernel(encoder, W, v_i, v_j):
    new_node = jnp.take(jnp.asarray(_LEAF), jnp.asarray(v_i, jnp.int32))
    lev = jnp.arange(_NLEV, dtype=jnp.int32)
    s = (_MAX_BITS - 2) - lev
    parent = lax.shift_right_logical(new_node, s + 1)
    child = lax.shift_right_logical(new_node, s)
    mult = jnp.where(child % 2 == 0, 1.0, -1.0).astype(jnp.float32)
    valid = (parent >= 2).astype(jnp.float32)
    pad = _NPAD - _NLEV
    parents = jnp.concatenate([parent, jnp.zeros((pad,), jnp.int32)])
    mult = jnp.concatenate([mult, jnp.zeros((pad,), jnp.float32)])
    valid = jnp.concatenate([valid, jnp.zeros((pad,), jnp.float32)])

    vj = jnp.asarray(v_j, jnp.int32)
    wbase = (parents >> 7) << 7
    wlane = parents & 127
    ebase = (vj >> 7) << 7
    elane = vj & 127
    lhm = elane & 15

    idx_row0 = jnp.zeros((_NSUB, 128), jnp.int32)
    idx_row0 = idx_row0.at[:, 0].set(wbase[:_NSUB])
    idx_row0 = idx_row0.at[:, 1].set(wbase[_NSUB:])
    idx_row0 = idx_row0.at[:, 2].set(ebase)
    idx_row0 = idx_row0.at[:, 3].set((elane >> 4) << 4)
    idx_row0 = idx_row0.at[:, 4].set((wlane[:_NSUB] >> 4) << 4)
    idx_row0 = idx_row0.at[:, 5].set((wlane[_NSUB:] >> 4) << 4)
    idx_row0 = idx_row0.at[:, 6].set(wlane[:_NSUB] & 15)
    idx_row0 = idx_row0.at[:, 7].set(wlane[_NSUB:] & 15)
    perm_rows = jnp.broadcast_to(
        jnp.asarray(_PERM_TBL)[None], (_NSUB, 4, 128))

    ohh16 = jax.nn.one_hot(lhm, 16, dtype=jnp.float32)
    row5 = jnp.zeros((_NSUB, 128), jnp.float32)
    row5 = row5.at[:, 0:16].set(jnp.broadcast_to(ohh16[None], (_NSUB, 16)))
    row6 = jnp.zeros((_NSUB, 128), jnp.float32)
    row6 = row6.at[:, 0].set(mult[:_NSUB])
    row6 = row6.at[:, 1].set(valid[:_NSUB])
    row6 = row6.at[:, 2].set(mult[_NSUB:])
    row6 = row6.at[:, 3].set(valid[_NSUB:])
    par3 = jnp.concatenate(
        [row5[:, None, :], row6[:, None, :],
         jnp.zeros((_NSUB, 6, 128), jnp.float32)], axis=1)
    idx3 = jnp.concatenate(
        [idx_row0[:, None, :], perm_rows,
         jnp.zeros((_NSUB, 3, 128), jnp.int32)], axis=1)

    out = _hs_path_kernel(W.T, encoder.T, idx3, par3)
    return out[0:1]


# speedup vs baseline: 11.0213x; 1.5973x over previous
import functools

import numpy as np
import jax
import jax.numpy as jnp
from jax import lax
from jax.experimental import pallas as pl
from jax.experimental.pallas import tpu as pltpu
from jax.experimental.pallas import tpu_sc as plsc

_SIZE_VERTEX = 100000
_D = 64


def _build_leaves(size_vertex):
    leaf = []

    def rec(tl, tr, v):
        if tl == tr:
            leaf.append(v)
            return
        tm = (tl + tr) >> 1
        rec(tl, tm, 2 * v)
        rec(tm + 1, tr, 2 * v + 1)

    rec(1, size_vertex + 1, 1)
    return leaf


_LEAF = np.asarray(_build_leaves(_SIZE_VERTEX), dtype=np.int32)
_MAX_BITS = int(_LEAF.max()).bit_length()
_NLEV = _MAX_BITS - 1
_NPAD = 32
_NSUB = 16

_PERM_TBL = np.zeros((4, 128), dtype=np.int32)
for _k in range(4):
    _PERM_TBL[_k, :16] = np.arange(16, dtype=np.int32) ^ (1 << _k)

_mesh = plsc.VectorSubcoreMesh(
    core_axis_name="c", subcore_axis_name="s", num_cores=1)


@functools.partial(
    pl.kernel,
    out_type=jax.ShapeDtypeStruct((16,), jnp.float32),
    mesh=_mesh,
    compiler_params=pltpu.CompilerParams(use_tc_tiling_on_sc=True),
    scratch_types=[
        pltpu.VMEM((8, 128), jnp.int32),
        pltpu.VMEM((8, 128), jnp.float32),
        pltpu.VMEM((_D, 128), jnp.float32),
        pltpu.VMEM((_D, 128), jnp.float32),
        pltpu.VMEM((_D, 128), jnp.float32),
        pltpu.VMEM((16,), jnp.float32),
        pltpu.VMEM((_NPAD, 16), jnp.float32),
        pltpu.VMEM_SHARED((_NPAD, 16), jnp.float32),
        pltpu.SemaphoreType.DMA,
    ],
)
def _hs_path_kernel(wt_hbm, enct_hbm, idx3_hbm, par3_hbm, out_hbm,
                    idxt_v, part_v, w0_v, w1_v, e_v, fac_v, allfac_v,
                    shared, sem):
    cid = lax.axis_index("c")
    sid = lax.axis_index("s")

    @pl.when(cid == 0)
    def _():
        pltpu.sync_copy(idx3_hbm.at[sid], idxt_v)
        pltpu.sync_copy(par3_hbm.at[sid], part_v)
        ivec = idxt_v[0, pl.ds(0, 16)]
        wb0 = pl.multiple_of(ivec[0], 128)
        wb1 = pl.multiple_of(ivec[1], 128)
        eb = pl.multiple_of(ivec[2], 128)
        choh = pl.multiple_of(ivec[3], 16)
        ch0 = pl.multiple_of(ivec[4], 16)
        ch1 = pl.multiple_of(ivec[5], 16)
        lane0 = jnp.broadcast_to(ivec[6], (16,))
        lane1 = jnp.broadcast_to(ivec[7], (16,))
        perms = [idxt_v[1 + k, pl.ds(0, 16)] for k in range(4)]

        cp0 = pltpu.async_copy(wt_hbm.at[:, pl.ds(wb0, 128)], w0_v, sem)
        cp1 = pltpu.async_copy(wt_hbm.at[:, pl.ds(wb1, 128)], w1_v, sem)
        cpe = pltpu.async_copy(enct_hbm.at[:, pl.ds(eb, 128)], e_v, sem)
        cp0.wait()
        cp1.wait()
        cpe.wait()

        d0 = None
        d1 = None
        for c in range(_D):
            hc = e_v[c, pl.ds(choh, 16)]
            w0c = w0_v[c, pl.ds(ch0, 16)]
            w1c = w1_v[c, pl.ds(ch1, 16)]
            w0b = w0c.at[lane0].get(mode="promise_in_bounds")
            w1b = w1c.at[lane1].get(mode="promise_in_bounds")
            t0 = hc * w0b
            t1 = hc * w1b
            d0 = t0 if d0 is None else d0 + t0
            d1 = t1 if d1 is None else d1 + t1

        ohh = part_v[0, pl.ds(0, 16)]
        d0 = d0 * ohh
        d1 = d1 * ohh
        for k in range(4):
            d0 = d0 + d0.at[perms[k]].get(mode="promise_in_bounds")
            d1 = d1 + d1.at[perms[k]].get(mode="promise_in_bounds")

        pvec = part_v[1, pl.ds(0, 16)]
        m0, va0, m1, va1 = pvec[0], pvec[1], pvec[2], pvec[3]
        f0 = va0 * (1.0 / (1.0 + jnp.exp(-(m0 * d0)))) + (1.0 - va0)
        f1 = va1 * (1.0 / (1.0 + jnp.exp(-(m1 * d1)))) + (1.0 - va1)

        fac_v[pl.ds(0, 16)] = f0
        pltpu.sync_copy(fac_v, shared.at[sid])
        fac_v[pl.ds(0, 16)] = f1
        pltpu.sync_copy(fac_v, shared.at[sid + _NSUB])
        plsc.subcore_barrier()

        @pl.when(sid == 0)
        def _():
            pltpu.sync_copy(shared, allfac_v)
            p = allfac_v[0, pl.ds(0, 16)][0]
            for r in range(1, _NPAD):
                p = p * allfac_v[r, pl.ds(0, 16)][0]
            fac_v[pl.ds(0, 16)] = jnp.broadcast_to(p, (16,))
            pltpu.sync_copy(fac_v, out_hbm)


def kernel(encoder, W, v_i, v_j):
    new_node = jnp.take(jnp.asarray(_LEAF), jnp.asarray(v_i, jnp.int32))
    vj = jnp.asarray(v_j, jnp.int32)

    shp = (_NSUB, 8, 128)
    sub = lax.broadcasted_iota(jnp.int32, shp, 0)
    row = lax.broadcasted_iota(jnp.int32, shp, 1)
    lane = lax.broadcasted_iota(jnp.int32, shp, 2)

    lev_a = sub
    lev_b = sub + _NSUB
    sh_a = (_MAX_BITS - 1) - lev_a
    par_a = lax.shift_right_logical(new_node, sh_a)
    chd_a = lax.shift_right_logical(new_node, sh_a - 1)
    is_b = lev_b < _NLEV
    par_b = jnp.where(is_b, lax.shift_right_logical(new_node, 1), 0)
    chd_b = jnp.where(is_b, new_node, 0)

    eb = (vj >> 7) << 7
    choh = ((vj & 127) >> 4) << 4
    lhm = vj & 15

    perm = lane ^ (1 << jnp.maximum(row - 1, 0))
    row0 = jnp.select(
        [lane == 0, lane == 1, lane == 2, lane == 3,
         lane == 4, lane == 5, lane == 6, lane == 7],
        [(par_a >> 7) << 7, (par_b >> 7) << 7,
         jnp.broadcast_to(eb, shp), jnp.broadcast_to(choh, shp),
         ((par_a & 127) >> 4) << 4, ((par_b & 127) >> 4) << 4,
         par_a & 15, par_b & 15],
        0)
    idx3 = jnp.where(
        row == 0, row0,
        jnp.where((row >= 1) & (row <= 4) & (lane < 16), perm, 0))

    mult_a = jnp.where(chd_a % 2 == 0, 1.0, -1.0)
    mult_b = jnp.where(chd_b % 2 == 0, 1.0, -1.0)
    valid_a = (par_a >= 2).astype(jnp.float32)
    valid_b = ((par_b >= 2) & is_b).astype(jnp.float32)
    par_row1 = jnp.select(
        [lane == 0, lane == 1, lane == 2, lane == 3],
        [mult_a, valid_a, mult_b, valid_b],
        0.0)
    par3 = jnp.where(
        (row == 0) & (lane == lhm), 1.0,
        jnp.where(row == 1, par_row1, 0.0))

    out = _hs_path_kernel(W.T, encoder.T, idx3, par3)
    return out[0:1]
